# Initial kernel scaffold; baseline (speedup 1.0000x reference)
import jax
import jax.numpy as jnp
from jax.experimental import pallas as pl


def kernel(input, codebooks):
    raise NotImplementedError("write your pallas kernel here")


# speedup vs baseline: 1.9937x; 1.9937x over previous
import functools

import jax
import jax.numpy as jnp
from jax.experimental import pallas as pl

_NUM_BOOK = 8
_NUM_CLUSTER = 1024
_D = 32
_BN = 2048


def _rq_body(x_ref, cb_ref, out_ref):
    x = x_ref[...]
    x2 = jnp.sum(x * x, axis=1, keepdims=True)
    cb = cb_ref[...]
    c2 = jnp.sum(cb * cb, axis=1)[None, :]
    scores = jax.lax.dot_general(
        x, cb, (((1,), (1,)), ((), ())),
        preferred_element_type=jnp.float32,
    )
    d2 = (x2 + c2) - 2.0 * scores
    cols = []
    iota = jax.lax.broadcasted_iota(jnp.int32, (x.shape[0], _NUM_CLUSTER), 1)
    for i in range(_NUM_BOOK):
        blk = d2[:, i * _NUM_CLUSTER:(i + 1) * _NUM_CLUSTER]
        mn = jnp.min(blk, axis=1, keepdims=True)
        idx = jnp.min(
            jnp.where(blk == mn, iota, _NUM_CLUSTER), axis=1, keepdims=True
        )
        cols.append(idx)
    out_ref[...] = jnp.concatenate(cols, axis=1).astype(jnp.int32)


@jax.jit
def kernel(input, codebooks):
    n, d = input.shape
    cb = codebooks.reshape(_NUM_BOOK * _NUM_CLUSTER, d)
    grid = (n // _BN,)
    return pl.pallas_call(
        _rq_body,
        grid=grid,
        in_specs=[
            pl.BlockSpec((_BN, d), lambda i: (i, 0)),
            pl.BlockSpec((_NUM_BOOK * _NUM_CLUSTER, d), lambda i: (0, 0)),
        ],
        out_specs=pl.BlockSpec((_BN, _NUM_BOOK), lambda i: (i, 0)),
        out_shape=jax.ShapeDtypeStruct((n, _NUM_BOOK), jnp.int32),
    )(input, cb)
